# Initial kernel scaffold; baseline (speedup 1.0000x reference)
import jax
import jax.numpy as jnp
from jax.experimental import pallas as pl


def kernel(lv_coarse, neighbor_idx, gn_gamma, gn_beta, weight):
    raise NotImplementedError("write your pallas kernel here")


# speedup vs baseline: 5.7401x; 5.7401x over previous
import functools

import jax
import jax.numpy as jnp
from jax import lax
from jax.experimental import pallas as pl
from jax.experimental.pallas import tpu as pltpu
from jax.experimental.pallas import tpu_sc as plsc

N_COARSE = 12500
N_FINE = 25000
D = 256
K = 9
CG = 8
EPS = 1e-5

RB = 512
NCP = 12800
NRB = NCP // RB

NW = 32
L = 16
SC_C = 200
NCHUNK = N_FINE // SC_C
CPW = -(-NCHUNK // NW)


def _tc_body(x_ref, g_ref, b_ref, w_ref, out_ref, sb_ref):
    i = pl.program_id(0)

    @pl.when(i == 0)
    def _stats():
        x = x_ref[...]
        s = jnp.sum(x, axis=0, keepdims=True)
        q = jnp.sum(x * x, axis=0, keepdims=True)
        row = lax.broadcasted_iota(jnp.int32, (D, D), 0) // CG
        col = lax.broadcasted_iota(jnp.int32, (D, D), 1) // CG
        m = (row == col).astype(jnp.float32)
        cnt = float(CG * N_COARSE)
        gs = jnp.dot(s, m, preferred_element_type=jnp.float32) / cnt
        gq = jnp.dot(q, m, preferred_element_type=jnp.float32) / cnt
        var = gq - gs * gs
        scale = g_ref[...] * lax.rsqrt(var + EPS)
        bias = b_ref[...] - gs * scale
        sb_ref[0:1, :] = scale
        sb_ref[1:2, :] = bias

    scale = sb_ref[0:1, :]
    bias = sb_ref[1:2, :]
    x = x_ref[pl.ds(i * RB, RB), :]
    y = jnp.maximum(x * scale + bias, 0.0)
    for k in range(K):
        out_ref[:, k * D:(k + 1) * D] = jnp.dot(
            y, w_ref[pl.ds(k * D, D), :], preferred_element_type=jnp.float32)


IROWS = 1760
IPAD = IROWS * 128


def _idx_body(i_ref, o_ref):
    pos = (lax.broadcasted_iota(jnp.int32, (IROWS, 128), 0) * 128
           + lax.broadcasted_iota(jnp.int32, (IROWS, 128), 1))
    o_ref[...] = i_ref[...] * K + pos // N_FINE


def _sc_body(tab_hbm, idxt_hbm, out_hbm, idxb, gbuf, acc, sem):
    wid = lax.axis_index("s") * 2 + lax.axis_index("c")

    def chunk_body(ci, carry):
        chunk = wid + ci * NW

        @pl.when(chunk < NCHUNK)
        def _():
            base = chunk * SC_C
            for k in range(K):
                pltpu.sync_copy(idxt_hbm.at[pl.ds(k * N_FINE + base, SC_C)],
                                idxb)
                dst = acc if k == 0 else gbuf
                pltpu.async_copy(tab_hbm.at[idxb], dst, sem).wait()
                if k > 0:
                    def _accrow(r, c):
                        for cc in range(D // L):
                            plsc.addupdate(acc.at[r, pl.ds(cc * L, L)],
                                           gbuf[r, pl.ds(cc * L, L)])
                        return c

                    lax.fori_loop(0, SC_C, _accrow, 0)
            pltpu.sync_copy(acc, out_hbm.at[pl.ds(base, SC_C)])

        return carry

    lax.fori_loop(0, CPW, chunk_body, 0)


def kernel(lv_coarse, neighbor_idx, gn_gamma, gn_beta, weight):
    lv_pad = jnp.pad(lv_coarse, ((0, NCP - N_COARSE), (0, 0)))
    table = pl.pallas_call(
        _tc_body,
        grid=(NRB,),
        in_specs=[
            pl.BlockSpec((NCP, D), lambda i: (0, 0)),
            pl.BlockSpec((1, D), lambda i: (0, 0)),
            pl.BlockSpec((1, D), lambda i: (0, 0)),
            pl.BlockSpec((K * D, D), lambda i: (0, 0)),
        ],
        out_specs=pl.BlockSpec((RB, K * D), lambda i: (i, 0)),
        out_shape=jax.ShapeDtypeStruct((NCP, K * D), jnp.float32),
        scratch_shapes=[pltpu.VMEM((2, D), jnp.float32)],
    )(lv_pad, gn_gamma.reshape(1, D), gn_beta.reshape(1, D), weight)

    tab_flat = table.reshape(NCP * K, D)
    idxt = neighbor_idx.T.reshape(-1)
    idxt_pad = jnp.pad(idxt, (0, IPAD - K * N_FINE)).reshape(IROWS, 128)
    flat_idx = pl.pallas_call(
        _idx_body,
        out_shape=jax.ShapeDtypeStruct((IROWS, 128), jnp.int32),
    )(idxt_pad).reshape(IPAD)

    mesh = plsc.VectorSubcoreMesh(core_axis_name="c", subcore_axis_name="s")
    out = pl.kernel(
        _sc_body,
        out_type=jax.ShapeDtypeStruct((N_FINE, D), jnp.float32),
        mesh=mesh,
        scratch_types=[
            pltpu.VMEM((SC_C,), jnp.int32),
            pltpu.VMEM((SC_C, D), jnp.float32),
            pltpu.VMEM((SC_C, D), jnp.float32),
            pltpu.SemaphoreType.DMA,
        ],
    )(tab_flat, flat_idx)
    return out


# speedup vs baseline: 6.4877x; 1.1302x over previous
import functools

import jax
import jax.numpy as jnp
from jax import lax
from jax.experimental import pallas as pl
from jax.experimental.pallas import tpu as pltpu
from jax.experimental.pallas import tpu_sc as plsc

N_COARSE = 12500
N_FINE = 25000
D = 256
K = 9
CG = 8
EPS = 1e-5

RB = 512
NCP = 12800
NRB = NCP // RB

NW = 32
L = 16
SC_C = 120
NCHUNK = -(-N_FINE // SC_C)
TAIL = N_FINE - (NCHUNK - 1) * SC_C
CPW = -(-NCHUNK // NW)

NIDX = NCHUNK * SC_C * K
IROWS = -(-NIDX // 128)
IPAD = IROWS * 128


def _tc_body(x_ref, g_ref, b_ref, w_ref, i2_ref, out_ref, o2_ref, sb_ref):
    i = pl.program_id(0)

    @pl.when(i == 0)
    def _stats():
        x = x_ref[...]
        s = jnp.sum(x, axis=0, keepdims=True)
        q = jnp.sum(x * x, axis=0, keepdims=True)
        row = lax.broadcasted_iota(jnp.int32, (D, D), 0) // CG
        col = lax.broadcasted_iota(jnp.int32, (D, D), 1) // CG
        m = (row == col).astype(jnp.float32)
        cnt = float(CG * N_COARSE)
        gs = jnp.dot(s, m, preferred_element_type=jnp.float32) / cnt
        gq = jnp.dot(q, m, preferred_element_type=jnp.float32) / cnt
        var = gq - gs * gs
        scale = g_ref[...] * lax.rsqrt(var + EPS)
        bias = b_ref[...] - gs * scale
        sb_ref[0:1, :] = scale
        sb_ref[1:2, :] = bias
        pos = (lax.broadcasted_iota(jnp.int32, (IROWS, 128), 0) * 128
               + lax.broadcasted_iota(jnp.int32, (IROWS, 128), 1))
        o2_ref[...] = i2_ref[...] * K + (pos // SC_C) % K

    scale = sb_ref[0:1, :]
    bias = sb_ref[1:2, :]
    x = x_ref[pl.ds(i * RB, RB), :]
    y = jnp.maximum(x * scale + bias, 0.0)
    for k in range(K):
        out_ref[:, k * D:(k + 1) * D] = jnp.dot(
            y, w_ref[pl.ds(k * D, D), :], preferred_element_type=jnp.float32)


def _sc_body(tab_hbm, idx_hbm, out_hbm, ib, gb0, gb1, acc, sem0, sem1):
    wid = lax.axis_index("s") * 2 + lax.axis_index("c")
    gbs = (gb0, gb1)
    sems = (sem0, sem1)

    def _accum(buf):
        def _row(r, c):
            for cc in range(D // L):
                plsc.addupdate(acc.at[r, pl.ds(cc * L, L)],
                               buf[r, pl.ds(cc * L, L)])
            return c

        lax.fori_loop(0, SC_C, _row, 0)

    def chunk_body(ci, carry):
        chunk = wid + ci * NW

        @pl.when(chunk < NCHUNK)
        def _():
            for k in range(K):
                pltpu.async_copy(
                    idx_hbm.at[pl.ds(chunk * K * SC_C + k * SC_C, SC_C)],
                    ib.at[k], sem0)
            for k in range(K):
                pltpu.make_async_copy(
                    idx_hbm.at[pl.ds(chunk * K * SC_C + k * SC_C, SC_C)],
                    ib.at[k], sem0).wait()
            pltpu.async_copy(tab_hbm.at[ib.at[0]], acc, sem0)
            pltpu.async_copy(tab_hbm.at[ib.at[1]], gb1, sem1)
            pltpu.make_async_copy(tab_hbm.at[ib.at[0]], acc, sem0).wait()
            for k in range(1, K):
                b = k % 2
                pltpu.make_async_copy(tab_hbm.at[ib.at[k]], gbs[b],
                                      sems[b]).wait()
                if k + 1 < K:
                    nb = (k + 1) % 2
                    pltpu.async_copy(tab_hbm.at[ib.at[k + 1]], gbs[nb],
                                     sems[nb])
                _accum(gbs[b])
            base = chunk * SC_C

            @pl.when(chunk < NCHUNK - 1)
            def _full():
                pltpu.sync_copy(acc, out_hbm.at[pl.ds(base, SC_C)])

            @pl.when(chunk == NCHUNK - 1)
            def _tail():
                pltpu.sync_copy(acc.at[pl.ds(0, TAIL)],
                                out_hbm.at[pl.ds(base, TAIL)])

        return carry

    lax.fori_loop(0, CPW, chunk_body, 0)


def kernel(lv_coarse, neighbor_idx, gn_gamma, gn_beta, weight):
    lv_pad = jnp.pad(lv_coarse, ((0, NCP - N_COARSE), (0, 0)))
    idx_pad = jnp.pad(neighbor_idx, ((0, NCHUNK * SC_C - N_FINE), (0, 0)))
    idx_cm = idx_pad.reshape(NCHUNK, SC_C, K).transpose(0, 2, 1).reshape(-1)
    idx_2d = jnp.pad(idx_cm, (0, IPAD - NIDX)).reshape(IROWS, 128)

    table, flat_idx = pl.pallas_call(
        _tc_body,
        grid=(NRB,),
        in_specs=[
            pl.BlockSpec((NCP, D), lambda i: (0, 0)),
            pl.BlockSpec((1, D), lambda i: (0, 0)),
            pl.BlockSpec((1, D), lambda i: (0, 0)),
            pl.BlockSpec((K * D, D), lambda i: (0, 0)),
            pl.BlockSpec((IROWS, 128), lambda i: (0, 0)),
        ],
        out_specs=[
            pl.BlockSpec((RB, K * D), lambda i: (i, 0)),
            pl.BlockSpec((IROWS, 128), lambda i: (0, 0)),
        ],
        out_shape=[
            jax.ShapeDtypeStruct((NCP, K * D), jnp.float32),
            jax.ShapeDtypeStruct((IROWS, 128), jnp.int32),
        ],
        scratch_shapes=[pltpu.VMEM((2, D), jnp.float32)],
    )(lv_pad, gn_gamma.reshape(1, D), gn_beta.reshape(1, D), weight, idx_2d)

    tab_flat = table.reshape(NCP * K, D)
    flat_idx = flat_idx.reshape(IPAD)

    mesh = plsc.VectorSubcoreMesh(core_axis_name="c", subcore_axis_name="s")
    out = pl.kernel(
        _sc_body,
        out_type=jax.ShapeDtypeStruct((N_FINE, D), jnp.float32),
        mesh=mesh,
        scratch_types=[
            pltpu.VMEM((K, SC_C), jnp.int32),
            pltpu.VMEM((SC_C, D), jnp.float32),
            pltpu.VMEM((SC_C, D), jnp.float32),
            pltpu.VMEM((SC_C, D), jnp.float32),
            pltpu.SemaphoreType.DMA,
            pltpu.SemaphoreType.DMA,
        ],
    )(tab_flat, flat_idx)
    return out


# speedup vs baseline: 6.5137x; 1.0040x over previous
import functools

import jax
import jax.numpy as jnp
from jax import lax
from jax.experimental import pallas as pl
from jax.experimental.pallas import tpu as pltpu
from jax.experimental.pallas import tpu_sc as plsc

N_COARSE = 12500
N_FINE = 25000
D = 256
K = 9
CG = 8
EPS = 1e-5

RB = 512
NCP = 12800
NRB = NCP // RB

NW = 32
L = 16
SC_C = 112
NCHUNK = -(-N_FINE // SC_C)
TAIL = N_FINE - (NCHUNK - 1) * SC_C
CPW = -(-NCHUNK // NW)

NIDX = NCHUNK * SC_C * K
IROWS = -(-NIDX // 128)
IPAD = IROWS * 128


def _tc_body(x_ref, g_ref, b_ref, w_ref, i2_ref, out_ref, o2_ref, sb_ref):
    i = pl.program_id(0)

    @pl.when(i == 0)
    def _stats():
        x = x_ref[...]
        s = jnp.sum(x, axis=0, keepdims=True)
        q = jnp.sum(x * x, axis=0, keepdims=True)
        row = lax.broadcasted_iota(jnp.int32, (D, D), 0) // CG
        col = lax.broadcasted_iota(jnp.int32, (D, D), 1) // CG
        m = (row == col).astype(jnp.float32)
        cnt = float(CG * N_COARSE)
        gs = jnp.dot(s, m, preferred_element_type=jnp.float32) / cnt
        gq = jnp.dot(q, m, preferred_element_type=jnp.float32) / cnt
        var = gq - gs * gs
        scale = g_ref[...] * lax.rsqrt(var + EPS)
        bias = b_ref[...] - gs * scale
        sb_ref[0:1, :] = scale
        sb_ref[1:2, :] = bias
        pos = (lax.broadcasted_iota(jnp.int32, (IROWS, 128), 0) * 128
               + lax.broadcasted_iota(jnp.int32, (IROWS, 128), 1))
        o2_ref[...] = i2_ref[...] * K + (pos // SC_C) % K

    scale = sb_ref[0:1, :]
    bias = sb_ref[1:2, :]
    x = x_ref[pl.ds(i * RB, RB), :]
    y = jnp.maximum(x * scale + bias, 0.0)
    for k in range(K):
        out_ref[:, k * D:(k + 1) * D] = jnp.dot(
            y, w_ref[pl.ds(k * D, D), :], preferred_element_type=jnp.float32)


def _sc_body(tab_hbm, idx_hbm, out_hbm, ib, gb0, gb1, acc, sem0, sem1):
    wid = lax.axis_index("s") * 2 + lax.axis_index("c")
    gbs = (gb0, gb1)
    sems = (sem0, sem1)

    def _accum(buf):
        def _row(r, c):
            for cc in range(D // L):
                plsc.addupdate(acc.at[r, pl.ds(cc * L, L)],
                               buf[r, pl.ds(cc * L, L)])
            return c

        lax.fori_loop(0, SC_C, _row, 0)

    def chunk_body(ci, carry):
        chunk = wid + ci * NW

        @pl.when(chunk < NCHUNK)
        def _():
            for k in range(K):
                pltpu.async_copy(
                    idx_hbm.at[pl.ds(chunk * K * SC_C + k * SC_C, SC_C)],
                    ib.at[k], sem0)
            for k in range(K):
                pltpu.make_async_copy(
                    idx_hbm.at[pl.ds(chunk * K * SC_C + k * SC_C, SC_C)],
                    ib.at[k], sem0).wait()
            pltpu.async_copy(tab_hbm.at[ib.at[0]], acc, sem0)
            pltpu.async_copy(tab_hbm.at[ib.at[1]], gb1, sem1)
            pltpu.make_async_copy(tab_hbm.at[ib.at[0]], acc, sem0).wait()
            for k in range(1, K):
                b = k % 2
                pltpu.make_async_copy(tab_hbm.at[ib.at[k]], gbs[b],
                                      sems[b]).wait()
                if k + 1 < K:
                    nb = (k + 1) % 2
                    pltpu.async_copy(tab_hbm.at[ib.at[k + 1]], gbs[nb],
                                     sems[nb])
                _accum(gbs[b])
            base = chunk * SC_C

            @pl.when(chunk < NCHUNK - 1)
            def _full():
                pltpu.sync_copy(acc, out_hbm.at[pl.ds(base, SC_C)])

            @pl.when(chunk == NCHUNK - 1)
            def _tail():
                pltpu.sync_copy(acc.at[pl.ds(0, TAIL)],
                                out_hbm.at[pl.ds(base, TAIL)])

        return carry

    lax.fori_loop(0, CPW, chunk_body, 0)


def kernel(lv_coarse, neighbor_idx, gn_gamma, gn_beta, weight):
    lv_pad = jnp.pad(lv_coarse, ((0, NCP - N_COARSE), (0, 0)))
    idx_pad = jnp.pad(neighbor_idx, ((0, NCHUNK * SC_C - N_FINE), (0, 0)))
    idx_cm = idx_pad.reshape(NCHUNK, SC_C, K).transpose(0, 2, 1).reshape(-1)
    idx_2d = jnp.pad(idx_cm, (0, IPAD - NIDX)).reshape(IROWS, 128)

    table, flat_idx = pl.pallas_call(
        _tc_body,
        grid=(NRB,),
        in_specs=[
            pl.BlockSpec((NCP, D), lambda i: (0, 0)),
            pl.BlockSpec((1, D), lambda i: (0, 0)),
            pl.BlockSpec((1, D), lambda i: (0, 0)),
            pl.BlockSpec((K * D, D), lambda i: (0, 0)),
            pl.BlockSpec((IROWS, 128), lambda i: (0, 0)),
        ],
        out_specs=[
            pl.BlockSpec((RB, K * D), lambda i: (i, 0)),
            pl.BlockSpec((IROWS, 128), lambda i: (0, 0)),
        ],
        out_shape=[
            jax.ShapeDtypeStruct((NCP, K * D), jnp.float32),
            jax.ShapeDtypeStruct((IROWS, 128), jnp.int32),
        ],
        scratch_shapes=[pltpu.VMEM((2, D), jnp.float32)],
    )(lv_pad, gn_gamma.reshape(1, D), gn_beta.reshape(1, D), weight, idx_2d)

    tab_flat = table.reshape(NCP * K, D)
    flat_idx = flat_idx.reshape(IPAD)

    mesh = plsc.VectorSubcoreMesh(core_axis_name="c", subcore_axis_name="s")
    out = pl.kernel(
        _sc_body,
        out_type=jax.ShapeDtypeStruct((N_FINE, D), jnp.float32),
        mesh=mesh,
        scratch_types=[
            pltpu.VMEM((K, SC_C), jnp.int32),
            pltpu.VMEM((SC_C, D), jnp.float32),
            pltpu.VMEM((SC_C, D), jnp.float32),
            pltpu.VMEM((SC_C, D), jnp.float32),
            pltpu.SemaphoreType.DMA,
            pltpu.SemaphoreType.DMA,
        ],
    )(tab_flat, flat_idx)
    return out


# speedup vs baseline: 7.0637x; 1.0844x over previous
import functools

import jax
import jax.numpy as jnp
from jax import lax
from jax.experimental import pallas as pl
from jax.experimental.pallas import tpu as pltpu
from jax.experimental.pallas import tpu_sc as plsc

N_COARSE = 12500
N_FINE = 25000
D = 256
K = 9
CG = 8
EPS = 1e-5

RB = 512
NCP = 12800
NRB = NCP // RB

NW = 32
L = 16
SC_C = 112
NCHUNK = -(-N_FINE // SC_C)
TAIL = N_FINE - (NCHUNK - 1) * SC_C
CPW = -(-NCHUNK // NW)

NIDX = NCHUNK * SC_C * K
IROWS = -(-NIDX // 128)
IPAD = IROWS * 128


def _tc_body(x_ref, g_ref, b_ref, w_ref, i2_ref, out_ref, o2_ref, sb_ref,
             y_ref):
    i = pl.program_id(0)
    k = pl.program_id(1)

    @pl.when((i == 0) & (k == 0))
    def _stats():
        x = x_ref[...]
        s = jnp.sum(x, axis=0, keepdims=True)
        q = jnp.sum(x * x, axis=0, keepdims=True)
        row = lax.broadcasted_iota(jnp.int32, (D, D), 0) // CG
        col = lax.broadcasted_iota(jnp.int32, (D, D), 1) // CG
        m = (row == col).astype(jnp.float32)
        cnt = float(CG * N_COARSE)
        gs = jnp.dot(s, m, preferred_element_type=jnp.float32) / cnt
        gq = jnp.dot(q, m, preferred_element_type=jnp.float32) / cnt
        var = gq - gs * gs
        scale = g_ref[...] * lax.rsqrt(var + EPS)
        bias = b_ref[...] - gs * scale
        sb_ref[0:1, :] = scale
        sb_ref[1:2, :] = bias
        pos = (lax.broadcasted_iota(jnp.int32, (IROWS, 128), 0) * 128
               + lax.broadcasted_iota(jnp.int32, (IROWS, 128), 1))
        o2_ref[...] = i2_ref[...] + ((pos // SC_C) % K) * NCP

    @pl.when(k == 0)
    def _norm():
        scale = sb_ref[0:1, :]
        bias = sb_ref[1:2, :]
        x = x_ref[pl.ds(i * RB, RB), :]
        y_ref[...] = jnp.maximum(x * scale + bias, 0.0)

    out_ref[0] = jnp.dot(y_ref[...], w_ref[pl.ds(k * D, D), :],
                         preferred_element_type=jnp.float32)


def _sc_body(tab_hbm, idx_hbm, out_hbm, ib, gb0, gb1, acc, sem0, sem1):
    wid = lax.axis_index("s") * 2 + lax.axis_index("c")
    gbs = (gb0, gb1)
    sems = (sem0, sem1)

    def _accum(buf):
        def _row(r, c):
            for cc in range(D // L):
                plsc.addupdate(acc.at[r, pl.ds(cc * L, L)],
                               buf[r, pl.ds(cc * L, L)])
            return c

        lax.fori_loop(0, SC_C, _row, 0)

    def chunk_body(ci, carry):
        chunk = wid + ci * NW

        @pl.when(chunk < NCHUNK)
        def _():
            for k in range(K):
                pltpu.async_copy(
                    idx_hbm.at[pl.ds(chunk * K * SC_C + k * SC_C, SC_C)],
                    ib.at[k], sem0)
            for k in range(K):
                pltpu.make_async_copy(
                    idx_hbm.at[pl.ds(chunk * K * SC_C + k * SC_C, SC_C)],
                    ib.at[k], sem0).wait()
            pltpu.async_copy(tab_hbm.at[ib.at[0]], acc, sem0)
            pltpu.async_copy(tab_hbm.at[ib.at[1]], gb1, sem1)
            pltpu.make_async_copy(tab_hbm.at[ib.at[0]], acc, sem0).wait()
            for k in range(1, K):
                b = k % 2
                pltpu.make_async_copy(tab_hbm.at[ib.at[k]], gbs[b],
                                      sems[b]).wait()
                if k + 1 < K:
                    nb = (k + 1) % 2
                    pltpu.async_copy(tab_hbm.at[ib.at[k + 1]], gbs[nb],
                                     sems[nb])
                _accum(gbs[b])
            base = chunk * SC_C

            @pl.when(chunk < NCHUNK - 1)
            def _full():
                pltpu.sync_copy(acc, out_hbm.at[pl.ds(base, SC_C)])

            @pl.when(chunk == NCHUNK - 1)
            def _tail():
                pltpu.sync_copy(acc.at[pl.ds(0, TAIL)],
                                out_hbm.at[pl.ds(base, TAIL)])

        return carry

    lax.fori_loop(0, CPW, chunk_body, 0)


def kernel(lv_coarse, neighbor_idx, gn_gamma, gn_beta, weight):
    lv_pad = jnp.pad(lv_coarse, ((0, NCP - N_COARSE), (0, 0)))
    idx_pad = jnp.pad(neighbor_idx, ((0, NCHUNK * SC_C - N_FINE), (0, 0)))
    idx_cm = idx_pad.reshape(NCHUNK, SC_C, K).transpose(0, 2, 1).reshape(-1)
    idx_2d = jnp.pad(idx_cm, (0, IPAD - NIDX)).reshape(IROWS, 128)

    table, flat_idx = pl.pallas_call(
        _tc_body,
        grid=(NRB, K),
        in_specs=[
            pl.BlockSpec((NCP, D), lambda i, k: (0, 0)),
            pl.BlockSpec((1, D), lambda i, k: (0, 0)),
            pl.BlockSpec((1, D), lambda i, k: (0, 0)),
            pl.BlockSpec((K * D, D), lambda i, k: (0, 0)),
            pl.BlockSpec((IROWS, 128), lambda i, k: (0, 0)),
        ],
        out_specs=[
            pl.BlockSpec((1, RB, D), lambda i, k: (k, i, 0)),
            pl.BlockSpec((IROWS, 128), lambda i, k: (0, 0)),
        ],
        out_shape=[
            jax.ShapeDtypeStruct((K, NCP, D), jnp.float32),
            jax.ShapeDtypeStruct((IROWS, 128), jnp.int32),
        ],
        scratch_shapes=[
            pltpu.VMEM((2, D), jnp.float32),
            pltpu.VMEM((RB, D), jnp.float32),
        ],
    )(lv_pad, gn_gamma.reshape(1, D), gn_beta.reshape(1, D), weight, idx_2d)

    tab_flat = table.reshape(K * NCP, D)
    flat_idx = flat_idx.reshape(IPAD)

    mesh = plsc.VectorSubcoreMesh(core_axis_name="c", subcore_axis_name="s")
    out = pl.kernel(
        _sc_body,
        out_type=jax.ShapeDtypeStruct((N_FINE, D), jnp.float32),
        mesh=mesh,
        scratch_types=[
            pltpu.VMEM((K, SC_C), jnp.int32),
            pltpu.VMEM((SC_C, D), jnp.float32),
            pltpu.VMEM((SC_C, D), jnp.float32),
            pltpu.VMEM((SC_C, D), jnp.float32),
            pltpu.SemaphoreType.DMA,
            pltpu.SemaphoreType.DMA,
        ],
    )(tab_flat, flat_idx)
    return out


# speedup vs baseline: 9.2134x; 1.3043x over previous
import functools

import jax
import jax.numpy as jnp
from jax import lax
from jax.experimental import pallas as pl
from jax.experimental.pallas import tpu as pltpu
from jax.experimental.pallas import tpu_sc as plsc

N_COARSE = 12500
N_FINE = 25000
D = 256
K = 9
CG = 8
EPS = 1e-5

RB = 512
NCP = 12800
NRB = NCP // RB

NW = 32
L = 16
SC_C = 112
NCHUNK = -(-N_FINE // SC_C)
TAIL = N_FINE - (NCHUNK - 1) * SC_C
CPW = -(-NCHUNK // NW)

NIDX = NCHUNK * SC_C * K
IROWS = -(-NIDX // 128)
IPAD = IROWS * 128


def _tc_body(x_ref, g_ref, b_ref, w_ref, i2_ref, out_ref, o2_ref, sb_ref):
    i = pl.program_id(0)

    @pl.when(i == 0)
    def _stats():
        x = x_ref[...]
        s = jnp.sum(x, axis=0, keepdims=True)
        q = jnp.sum(x * x, axis=0, keepdims=True)
        row = lax.broadcasted_iota(jnp.int32, (D, D), 0) // CG
        col = lax.broadcasted_iota(jnp.int32, (D, D), 1) // CG
        m = (row == col).astype(jnp.float32)
        cnt = float(CG * N_COARSE)
        gs = jnp.dot(s, m, preferred_element_type=jnp.float32) / cnt
        gq = jnp.dot(q, m, preferred_element_type=jnp.float32) / cnt
        var = gq - gs * gs
        scale = g_ref[...] * lax.rsqrt(var + EPS)
        bias = b_ref[...] - gs * scale
        sb_ref[0:1, :] = scale
        sb_ref[1:2, :] = bias
        pos = (lax.broadcasted_iota(jnp.int32, (IROWS, 128), 0) * 128
               + lax.broadcasted_iota(jnp.int32, (IROWS, 128), 1))
        o2_ref[...] = i2_ref[...] + ((pos // SC_C) % K) * NCP

    scale = sb_ref[0:1, :]
    bias = sb_ref[1:2, :]
    x = x_ref[pl.ds(i * RB, RB), :]
    y = jnp.maximum(x * scale + bias, 0.0)
    for k in range(K):
        out_ref[k] = jnp.dot(y, w_ref[pl.ds(k * D, D), :],
                             preferred_element_type=jnp.float32)


def _sc_body(tab_hbm, idx_hbm, out_hbm, ib, gb0, gb1, acc, sem0, sem1):
    wid = lax.axis_index("s") * 2 + lax.axis_index("c")
    gbs = (gb0, gb1)
    sems = (sem0, sem1)

    def _accum(buf):
        def _row(r, c):
            for cc in range(D // L):
                plsc.addupdate(acc.at[r, pl.ds(cc * L, L)],
                               buf[r, pl.ds(cc * L, L)])
            return c

        lax.fori_loop(0, SC_C, _row, 0)

    def chunk_body(ci, carry):
        chunk = wid + ci * NW

        @pl.when(chunk < NCHUNK)
        def _():
            for k in range(K):
                pltpu.async_copy(
                    idx_hbm.at[pl.ds(chunk * K * SC_C + k * SC_C, SC_C)],
                    ib.at[k], sem0)
            for k in range(K):
                pltpu.make_async_copy(
                    idx_hbm.at[pl.ds(chunk * K * SC_C + k * SC_C, SC_C)],
                    ib.at[k], sem0).wait()
            pltpu.async_copy(tab_hbm.at[ib.at[0]], acc, sem0)
            pltpu.async_copy(tab_hbm.at[ib.at[1]], gb1, sem1)
            pltpu.make_async_copy(tab_hbm.at[ib.at[0]], acc, sem0).wait()
            for k in range(1, K):
                b = k % 2
                pltpu.make_async_copy(tab_hbm.at[ib.at[k]], gbs[b],
                                      sems[b]).wait()
                if k + 1 < K:
                    nb = (k + 1) % 2
                    pltpu.async_copy(tab_hbm.at[ib.at[k + 1]], gbs[nb],
                                     sems[nb])
                _accum(gbs[b])
            base = chunk * SC_C

            @pl.when(chunk < NCHUNK - 1)
            def _full():
                pltpu.sync_copy(acc, out_hbm.at[pl.ds(base, SC_C)])

            @pl.when(chunk == NCHUNK - 1)
            def _tail():
                pltpu.sync_copy(acc.at[pl.ds(0, TAIL)],
                                out_hbm.at[pl.ds(base, TAIL)])

        return carry

    lax.fori_loop(0, CPW, chunk_body, 0)


def kernel(lv_coarse, neighbor_idx, gn_gamma, gn_beta, weight):
    lv_pad = jnp.pad(lv_coarse, ((0, NCP - N_COARSE), (0, 0)))
    idx_pad = jnp.pad(neighbor_idx, ((0, NCHUNK * SC_C - N_FINE), (0, 0)))
    idx_cm = idx_pad.reshape(NCHUNK, SC_C, K).transpose(0, 2, 1).reshape(-1)
    idx_2d = jnp.pad(idx_cm, (0, IPAD - NIDX)).reshape(IROWS, 128)

    table, flat_idx = pl.pallas_call(
        _tc_body,
        grid=(NRB,),
        in_specs=[
            pl.BlockSpec((NCP, D), lambda i: (0, 0)),
            pl.BlockSpec((1, D), lambda i: (0, 0)),
            pl.BlockSpec((1, D), lambda i: (0, 0)),
            pl.BlockSpec((K * D, D), lambda i: (0, 0)),
            pl.BlockSpec((IROWS, 128), lambda i: (0, 0)),
        ],
        out_specs=[
            pl.BlockSpec((K, RB, D), lambda i: (0, i, 0)),
            pl.BlockSpec((IROWS, 128), lambda i: (0, 0)),
        ],
        out_shape=[
            jax.ShapeDtypeStruct((K, NCP, D), jnp.float32),
            jax.ShapeDtypeStruct((IROWS, 128), jnp.int32),
        ],
        scratch_shapes=[
            pltpu.VMEM((2, D), jnp.float32),
        ],
    )(lv_pad, gn_gamma.reshape(1, D), gn_beta.reshape(1, D), weight, idx_2d)

    tab_flat = table.reshape(K * NCP, D)
    flat_idx = flat_idx.reshape(IPAD)

    mesh = plsc.VectorSubcoreMesh(core_axis_name="c", subcore_axis_name="s")
    out = pl.kernel(
        _sc_body,
        out_type=jax.ShapeDtypeStruct((N_FINE, D), jnp.float32),
        mesh=mesh,
        scratch_types=[
            pltpu.VMEM((K, SC_C), jnp.int32),
            pltpu.VMEM((SC_C, D), jnp.float32),
            pltpu.VMEM((SC_C, D), jnp.float32),
            pltpu.VMEM((SC_C, D), jnp.float32),
            pltpu.SemaphoreType.DMA,
            pltpu.SemaphoreType.DMA,
        ],
    )(tab_flat, flat_idx)
    return out
